# Initial kernel scaffold; baseline (speedup 1.0000x reference)
import math

import jax
import jax.numpy as jnp
from jax.experimental import pallas as pl

N = 10000
H = 64
TEMB = 64
B = 8
BLK = 1000
GRID = N // BLK

_LOG1E4 = math.log(10000.0)


def _silu(v):
    return v * jax.nn.sigmoid(v)


def _mm(a, b):
    return jax.lax.dot_general(a, b, (((1,), (0,)), ((), ())),
                               preferred_element_type=jnp.float32)


def _cond_body(t_ref, topo_ref, stab_ref, sust_ref,
               t1w_ref, t1b_ref, t2w_ref, t2b_ref,
               to1w_ref, to1b_ref, to2w_ref, to2b_ref,
               st1w_ref, st1b_ref, st2w_ref, st2b_ref,
               su1w_ref, su1b_ref, su2w_ref, su2b_ref,
               c1w_ref, c1b_ref, c2w_ref, c2b_ref,
               wv_ref, wow_ref, wob_ref,
               delta_ref):
    half = TEMB // 2
    freq = jnp.exp(jax.lax.broadcasted_iota(jnp.float32, (1, half), 1)
                   * (-_LOG1E4 / (half - 1)))
    ang = t_ref[...] * freq
    s, c = jnp.sin(ang), jnp.cos(ang)
    te_h = _silu(_mm(s, t1w_ref[:half, :]) + _mm(c, t1w_ref[half:, :])
                 + t1b_ref[...])
    temb = _mm(te_h, t2w_ref[...]) + t2b_ref[...]

    te = _mm(_silu(_mm(topo_ref[...], to1w_ref[...]) + to1b_ref[...]),
             to2w_ref[...]) + to2b_ref[...]
    se = _mm(_silu(_mm(stab_ref[...], st1w_ref[...]) + st1b_ref[...]),
             st2w_ref[...]) + st2b_ref[...]
    ue = _mm(_silu(_mm(sust_ref[...], su1w_ref[...]) + su1b_ref[...]),
             su2w_ref[...]) + su2b_ref[...]
    ce_h = _silu(_mm(te, c1w_ref[0:32, :]) + _mm(se, c1w_ref[32:48, :])
                 + _mm(ue, c1w_ref[48:64, :]) + c1b_ref[...])
    cond = _mm(ce_h, c2w_ref[...]) + c2b_ref[...]

    attn = _mm(_mm(cond, wv_ref[...]), wow_ref[...]) + wob_ref[...]
    delta_ref[...] = attn + _silu(temb)


def _node_body(x_ref, batch_ref, delta_ref,
               new_ref, neb_ref,
               np1w_ref, np1b_ref, np2w_ref, np2b_ref,
               pp1w_ref, pp1b_ref, pp2w_ref, pp2b_ref,
               node_out_ref, pos_out_ref):
    ids = batch_ref[0, 0, :].reshape(BLK, 1)
    onehot = (ids == jax.lax.broadcasted_iota(jnp.int32, (BLK, B), 1)
              ).astype(jnp.float32)
    u = _mm(onehot, delta_ref[...])
    h = _mm(x_ref[...], new_ref[...]) + neb_ref[...]
    h = h + u
    h = h + u
    h = h + u
    h = h + u
    a = _silu(_mm(h, np1w_ref[...]) + np1b_ref[...])
    node_out_ref[...] = _mm(a, np2w_ref[...]) + np2b_ref[...]
    g = _silu(_mm(h, pp1w_ref[...]) + pp1b_ref[...])
    pos_out_ref[...] = _mm(g, pp2w_ref[...]) + pp2b_ref[...]


def kernel(x, edge_index, edge_attr, pos, t, topo_cond, stab_cond, sust_cond,
           batch, params):
    del edge_index, edge_attr, pos
    p = params
    row = lambda b: b.reshape(1, -1)

    delta = pl.pallas_call(
        _cond_body,
        out_shape=jax.ShapeDtypeStruct((B, H), jnp.float32),
    )(t.reshape(B, 1),
      topo_cond, stab_cond, sust_cond,
      p['time1']['w'], row(p['time1']['b']),
      p['time2']['w'], row(p['time2']['b']),
      p['topo1']['w'], row(p['topo1']['b']),
      p['topo2']['w'], row(p['topo2']['b']),
      p['stab1']['w'], row(p['stab1']['b']),
      p['stab2']['w'], row(p['stab2']['b']),
      p['sust1']['w'], row(p['sust1']['b']),
      p['sust2']['w'], row(p['sust2']['b']),
      p['comb1']['w'], row(p['comb1']['b']),
      p['comb2']['w'], row(p['comb2']['b']),
      p['Wv'], p['Wo']['w'], row(p['Wo']['b']))

    batch3 = batch.reshape(GRID, 1, BLK)
    full = lambda shape: pl.BlockSpec(shape, lambda i: (0,) * len(shape))

    node_pred, pos_pred = pl.pallas_call(
        _node_body,
        grid=(GRID,),
        in_specs=[
            pl.BlockSpec((BLK, x.shape[1]), lambda i: (i, 0)),
            pl.BlockSpec((1, 1, BLK), lambda i: (i, 0, 0)),
            full((B, H)),
            full(p['node_emb']['w'].shape), full((1, H)),
            full(p['np1']['w'].shape), full((1, p['np1']['w'].shape[1])),
            full(p['np2']['w'].shape), full((1, p['np2']['w'].shape[1])),
            full(p['pp1']['w'].shape), full((1, p['pp1']['w'].shape[1])),
            full(p['pp2']['w'].shape), full((1, p['pp2']['w'].shape[1])),
        ],
        out_specs=[
            pl.BlockSpec((BLK, p['np2']['w'].shape[1]), lambda i: (i, 0)),
            pl.BlockSpec((BLK, p['pp2']['w'].shape[1]), lambda i: (i, 0)),
        ],
        out_shape=[
            jax.ShapeDtypeStruct((N, p['np2']['w'].shape[1]), jnp.float32),
            jax.ShapeDtypeStruct((N, p['pp2']['w'].shape[1]), jnp.float32),
        ],
    )(x, batch3, delta,
      p['node_emb']['w'], row(p['node_emb']['b']),
      p['np1']['w'], row(p['np1']['b']),
      p['np2']['w'], row(p['np2']['b']),
      p['pp1']['w'], row(p['pp1']['b']),
      p['pp2']['w'], row(p['pp2']['b']))

    return node_pred, pos_pred


# speedup vs baseline: 198.4194x; 198.4194x over previous
import math

import jax
import jax.numpy as jnp
from jax.experimental import pallas as pl

N = 10000
H = 64
TEMB = 64
B = 8
BLK = 1000
GRID = N // BLK

_LOG1E4 = math.log(10000.0)


def _silu(v):
    return v * jax.nn.sigmoid(v)


def _mm(a, b):
    return jax.lax.dot_general(a, b, (((1,), (0,)), ((), ())),
                               preferred_element_type=jnp.float32)


def _cond_body(t_ref, topo_ref, stab_ref, sust_ref,
               t1w_ref, t1b_ref, t2w_ref, t2b_ref,
               to1w_ref, to1b_ref, to2w_ref, to2b_ref,
               st1w_ref, st1b_ref, st2w_ref, st2b_ref,
               su1w_ref, su1b_ref, su2w_ref, su2b_ref,
               c1w_ref, c1b_ref, c2w_ref, c2b_ref,
               wv_ref, wow_ref, wob_ref,
               delta_ref):
    half = TEMB // 2
    freq = jnp.exp(jax.lax.broadcasted_iota(jnp.int32, (1, half), 1)
                   .astype(jnp.float32) * (-_LOG1E4 / (half - 1)))
    ang = t_ref[...] * freq
    s, c = jnp.sin(ang), jnp.cos(ang)
    te_h = _silu(_mm(s, t1w_ref[:half, :]) + _mm(c, t1w_ref[half:, :])
                 + t1b_ref[...])
    temb = _mm(te_h, t2w_ref[...]) + t2b_ref[...]

    te = _mm(_silu(_mm(topo_ref[...], to1w_ref[...]) + to1b_ref[...]),
             to2w_ref[...]) + to2b_ref[...]
    se = _mm(_silu(_mm(stab_ref[...], st1w_ref[...]) + st1b_ref[...]),
             st2w_ref[...]) + st2b_ref[...]
    ue = _mm(_silu(_mm(sust_ref[...], su1w_ref[...]) + su1b_ref[...]),
             su2w_ref[...]) + su2b_ref[...]
    ce_h = _silu(_mm(te, c1w_ref[0:32, :]) + _mm(se, c1w_ref[32:48, :])
                 + _mm(ue, c1w_ref[48:64, :]) + c1b_ref[...])
    cond = _mm(ce_h, c2w_ref[...]) + c2b_ref[...]

    attn = _mm(_mm(cond, wv_ref[...]), wow_ref[...]) + wob_ref[...]
    delta_ref[...] = attn + _silu(temb)


def _node_body(x_ref, batch_ref, delta_ref,
               new_ref, neb_ref,
               np1w_ref, np1b_ref, np2w_ref, np2b_ref,
               pp1w_ref, pp1b_ref, pp2w_ref, pp2b_ref,
               node_out_ref, pos_out_ref):
    ids = batch_ref[0, 0, :].reshape(BLK, 1)
    onehot = (ids == jax.lax.broadcasted_iota(jnp.int32, (BLK, B), 1)
              ).astype(jnp.float32)
    u = _mm(onehot, delta_ref[...])
    h = _mm(x_ref[...], new_ref[...]) + neb_ref[...]
    h = h + u
    h = h + u
    h = h + u
    h = h + u
    a = _silu(_mm(h, np1w_ref[...]) + np1b_ref[...])
    node_out_ref[...] = _mm(a, np2w_ref[...]) + np2b_ref[...]
    g = _silu(_mm(h, pp1w_ref[...]) + pp1b_ref[...])
    pos_out_ref[...] = _mm(g, pp2w_ref[...]) + pp2b_ref[...]


def kernel(x, edge_index, edge_attr, pos, t, topo_cond, stab_cond, sust_cond,
           batch, params):
    del edge_index, edge_attr, pos
    p = params
    row = lambda b: b.reshape(1, -1)

    delta = pl.pallas_call(
        _cond_body,
        out_shape=jax.ShapeDtypeStruct((B, H), jnp.float32),
    )(t.reshape(B, 1),
      topo_cond, stab_cond, sust_cond,
      p['time1']['w'], row(p['time1']['b']),
      p['time2']['w'], row(p['time2']['b']),
      p['topo1']['w'], row(p['topo1']['b']),
      p['topo2']['w'], row(p['topo2']['b']),
      p['stab1']['w'], row(p['stab1']['b']),
      p['stab2']['w'], row(p['stab2']['b']),
      p['sust1']['w'], row(p['sust1']['b']),
      p['sust2']['w'], row(p['sust2']['b']),
      p['comb1']['w'], row(p['comb1']['b']),
      p['comb2']['w'], row(p['comb2']['b']),
      p['Wv'], p['Wo']['w'], row(p['Wo']['b']))

    batch3 = batch.reshape(GRID, 1, BLK)
    full = lambda shape: pl.BlockSpec(shape, lambda i: (0,) * len(shape))

    node_pred, pos_pred = pl.pallas_call(
        _node_body,
        grid=(GRID,),
        in_specs=[
            pl.BlockSpec((BLK, x.shape[1]), lambda i: (i, 0)),
            pl.BlockSpec((1, 1, BLK), lambda i: (i, 0, 0)),
            full((B, H)),
            full(p['node_emb']['w'].shape), full((1, H)),
            full(p['np1']['w'].shape), full((1, p['np1']['w'].shape[1])),
            full(p['np2']['w'].shape), full((1, p['np2']['w'].shape[1])),
            full(p['pp1']['w'].shape), full((1, p['pp1']['w'].shape[1])),
            full(p['pp2']['w'].shape), full((1, p['pp2']['w'].shape[1])),
        ],
        out_specs=[
            pl.BlockSpec((BLK, p['np2']['w'].shape[1]), lambda i: (i, 0)),
            pl.BlockSpec((BLK, p['pp2']['w'].shape[1]), lambda i: (i, 0)),
        ],
        out_shape=[
            jax.ShapeDtypeStruct((N, p['np2']['w'].shape[1]), jnp.float32),
            jax.ShapeDtypeStruct((N, p['pp2']['w'].shape[1]), jnp.float32),
        ],
    )(x, batch3, delta,
      p['node_emb']['w'], row(p['node_emb']['b']),
      p['np1']['w'], row(p['np1']['b']),
      p['np2']['w'], row(p['np2']['b']),
      p['pp1']['w'], row(p['pp1']['b']),
      p['pp2']['w'], row(p['pp2']['b']))

    return node_pred, pos_pred


# speedup vs baseline: 203.0416x; 1.0233x over previous
import math

import jax
import jax.numpy as jnp
from jax.experimental import pallas as pl

N = 10000
H = 64
TEMB = 64
B = 8

_LOG1E4 = math.log(10000.0)


def _silu(v):
    return v * jax.nn.sigmoid(v)


def _mm(a, b):
    return jax.lax.dot_general(a, b, (((1,), (0,)), ((), ())),
                               preferred_element_type=jnp.float32)


def _body(x_ref, batch_ref, t_ref, topo_ref, stab_ref, sust_ref,
          t1w_ref, t1b_ref, t2w_ref, t2b_ref,
          to1w_ref, to1b_ref, to2w_ref, to2b_ref,
          st1w_ref, st1b_ref, st2w_ref, st2b_ref,
          su1w_ref, su1b_ref, su2w_ref, su2b_ref,
          c1w_ref, c1b_ref, c2w_ref, c2b_ref,
          wv_ref, wow_ref, wob_ref,
          new_ref, neb_ref,
          np1w_ref, np1b_ref, np2w_ref, np2b_ref,
          pp1w_ref, pp1b_ref, pp2w_ref, pp2b_ref,
          node_out_ref, pos_out_ref):
    half = TEMB // 2
    freq = jnp.exp(jax.lax.broadcasted_iota(jnp.int32, (1, half), 1)
                   .astype(jnp.float32) * (-_LOG1E4 / (half - 1)))
    ang = t_ref[...] * freq
    s, c = jnp.sin(ang), jnp.cos(ang)
    te_h = _silu(_mm(s, t1w_ref[:half, :]) + _mm(c, t1w_ref[half:, :])
                 + t1b_ref[...])
    temb = _mm(te_h, t2w_ref[...]) + t2b_ref[...]

    te = _mm(_silu(_mm(topo_ref[...], to1w_ref[...]) + to1b_ref[...]),
             to2w_ref[...]) + to2b_ref[...]
    se = _mm(_silu(_mm(stab_ref[...], st1w_ref[...]) + st1b_ref[...]),
             st2w_ref[...]) + st2b_ref[...]
    ue = _mm(_silu(_mm(sust_ref[...], su1w_ref[...]) + su1b_ref[...]),
             su2w_ref[...]) + su2b_ref[...]
    ce_h = _silu(_mm(te, c1w_ref[0:32, :]) + _mm(se, c1w_ref[32:48, :])
                 + _mm(ue, c1w_ref[48:64, :]) + c1b_ref[...])
    cond = _mm(ce_h, c2w_ref[...]) + c2b_ref[...]

    attn = _mm(_mm(cond, wv_ref[...]), wow_ref[...]) + wob_ref[...]
    delta = attn + _silu(temb)

    onehot = (batch_ref[...] ==
              jax.lax.broadcasted_iota(jnp.int32, (N, B), 1)
              ).astype(jnp.float32)
    u = _mm(onehot, delta)
    h = _mm(x_ref[...], new_ref[...]) + neb_ref[...]
    h = h + u
    h = h + u
    h = h + u
    h = h + u
    a = _silu(_mm(h, np1w_ref[...]) + np1b_ref[...])
    node_out_ref[...] = _mm(a, np2w_ref[...]) + np2b_ref[...]
    g = _silu(_mm(h, pp1w_ref[...]) + pp1b_ref[...])
    pos_out_ref[...] = _mm(g, pp2w_ref[...]) + pp2b_ref[...]


def kernel(x, edge_index, edge_attr, pos, t, topo_cond, stab_cond, sust_cond,
           batch, params):
    del edge_index, edge_attr, pos
    p = params
    row = lambda b: b.reshape(1, -1)

    node_pred, pos_pred = pl.pallas_call(
        _body,
        out_shape=[
            jax.ShapeDtypeStruct((N, p['np2']['w'].shape[1]), jnp.float32),
            jax.ShapeDtypeStruct((N, p['pp2']['w'].shape[1]), jnp.float32),
        ],
    )(x, batch.reshape(N, 1), t.reshape(B, 1),
      topo_cond, stab_cond, sust_cond,
      p['time1']['w'], row(p['time1']['b']),
      p['time2']['w'], row(p['time2']['b']),
      p['topo1']['w'], row(p['topo1']['b']),
      p['topo2']['w'], row(p['topo2']['b']),
      p['stab1']['w'], row(p['stab1']['b']),
      p['stab2']['w'], row(p['stab2']['b']),
      p['sust1']['w'], row(p['sust1']['b']),
      p['sust2']['w'], row(p['sust2']['b']),
      p['comb1']['w'], row(p['comb1']['b']),
      p['comb2']['w'], row(p['comb2']['b']),
      p['Wv'], p['Wo']['w'], row(p['Wo']['b']),
      p['node_emb']['w'], row(p['node_emb']['b']),
      p['np1']['w'], row(p['np1']['b']),
      p['np2']['w'], row(p['np2']['b']),
      p['pp1']['w'], row(p['pp1']['b']),
      p['pp2']['w'], row(p['pp2']['b']))

    return node_pred, pos_pred
